# baseline (speedup 1.0000x reference)
import math

import jax
import jax.numpy as jnp
import numpy as np
from jax import lax
from jax.experimental import pallas as pl
from jax.experimental.pallas import tpu as pltpu
from jax.experimental.pallas import tpu_sc as plsc

S = 4096
NUM_BUCKETS = 32
MAX_DISTANCE = 4096
NW = 32
RPW = S // NW
SPAN = 4224
GW = 8224
LANES = 16


def _bucket_table() -> np.ndarray:
    g = np.arange(GW, dtype=np.int64)
    rows = []
    for c in range(8):
        d = np.clip(g + c - (S - 1), -(S - 1), S - 1)
        a = np.abs(d)
        safe = np.maximum(a, 1).astype(np.float32)
        log_term = 8.0 + np.ceil(
            np.log(safe / 8.0) / math.log(MAX_DISTANCE / 8.0) * 8.0
        )
        large = np.minimum(np.float32(15.0), log_term).astype(np.int32)
        b = np.where(a < 8, a, large).astype(np.int32)
        rows.append(np.where(d < 0, b + 16, b).astype(np.int32))
    return np.stack(rows)


_BT = _bucket_table()


def _rpe_body(bt_hbm, w_hbm, out_hbm, *scratch):
    bt_v = scratch[0:8]
    v_v = scratch[8:16]
    w_v, sem = scratch[16], scratch[17]
    cid = lax.axis_index("c")
    sid = lax.axis_index("s")
    wid = sid * 2 + cid
    r0 = wid * RPW
    gbase = (S - RPW) - r0

    pltpu.sync_copy(w_hbm, w_v)
    for c in range(8):
        pltpu.sync_copy(bt_hbm.at[pl.ds(c * GW + gbase, SPAN)], bt_v[c])

    def gather_chunk(k, carry):
        for c in range(8):
            idx = bt_v[c][pl.ds(k * LANES, LANES)]
            v_v[c][pl.ds(k * LANES, LANES)] = plsc.load_gather(w_v, [idx])
        return carry

    lax.fori_loop(0, SPAN // LANES, gather_chunk, 0)

    def row_block(m, carry):
        o8 = (RPW - 8) - 8 * m
        handles = [
            pltpu.async_copy(
                v_v[c].at[pl.ds(o8, S)],
                out_hbm.at[r0 + (7 - c) + 8 * m],
                sem,
            )
            for c in range(8)
        ]
        for h in handles:
            h.wait()
        return carry

    lax.fori_loop(0, RPW // 8, row_block, 0)


def kernel(seq_len, weight):
    del seq_len
    w = weight.reshape(NUM_BUCKETS).astype(jnp.float32)
    run = pl.kernel(
        _rpe_body,
        out_type=jax.ShapeDtypeStruct((S, S), jnp.float32),
        mesh=plsc.VectorSubcoreMesh(core_axis_name="c", subcore_axis_name="s"),
        compiler_params=pltpu.CompilerParams(
            needs_layout_passes=False, use_tc_tiling_on_sc=False
        ),
        scratch_types=(
            [pltpu.VMEM((SPAN,), jnp.int32) for _ in range(8)]
            + [pltpu.VMEM((SPAN,), jnp.float32) for _ in range(8)]
            + [pltpu.VMEM((NUM_BUCKETS,), jnp.float32),
               pltpu.SemaphoreType.DMA]
        ),
    )
    return run(jnp.asarray(_BT).reshape(-1), w)


# speedup vs baseline: 1.7059x; 1.7059x over previous
import math

import jax
import jax.numpy as jnp
import numpy as np
from jax import lax
from jax.experimental import pallas as pl
from jax.experimental.pallas import tpu as pltpu
from jax.experimental.pallas import tpu_sc as plsc

S = 4096
NUM_BUCKETS = 32
MAX_DISTANCE = 4096
NW = 32
W = 6144
GW = 8320
LANES = 16
NSTRIPE = 16


def _bucket_table() -> np.ndarray:
    g = np.arange(GW, dtype=np.int64)
    rows = []
    for c in range(8):
        d = np.clip(g + c - (S - 1), -(S - 1), S - 1)
        a = np.abs(d)
        safe = np.maximum(a, 1).astype(np.float32)
        log_term = 8.0 + np.ceil(
            np.log(safe / 8.0) / math.log(MAX_DISTANCE / 8.0) * 8.0
        )
        large = np.minimum(np.float32(15.0), log_term).astype(np.int32)
        b = np.where(a < 8, a, large).astype(np.int32)
        rows.append(np.where(d < 0, b + 16, b).astype(np.int32))
    return np.stack(rows)


_BT = _bucket_table()


def _rpe_body(bt_hbm, w_hbm, out_hbm, *scratch):
    bt_v = scratch[0:8]
    b_v, w_v, sem = scratch[8], scratch[9], scratch[10]
    cid = lax.axis_index("c")
    sid = lax.axis_index("s")
    wid = sid * 2 + cid
    a = wid % 16
    h = wid // 16
    gb = (S - 8) - 8 * a - 128 * (NSTRIPE - 1) - 2048 * h

    pltpu.sync_copy(w_hbm, w_v)
    for c in range(8):
        pltpu.sync_copy(bt_hbm.at[pl.ds(c * GW + gb, W)], bt_v[c])

    def gather_chunk(k, carry):
        for c in range(8):
            idx = bt_v[c][pl.ds(k * LANES, LANES)]
            b_v[7 - c, pl.ds(k * LANES, LANES)] = plsc.load_gather(w_v, [idx])
        return carry

    lax.fori_loop(0, W // LANES, gather_chunk, 0)

    handles = [
        pltpu.async_copy(
            b_v.at[:, pl.ds(128 * (NSTRIPE - 1 - kk), S)],
            out_hbm.at[pl.ds(8 * a + 2048 * h + 128 * kk, 8), :],
            sem,
        )
        for kk in range(NSTRIPE)
    ]
    for hd in handles:
        hd.wait()


def kernel(seq_len, weight):
    del seq_len
    w = weight.reshape(NUM_BUCKETS).astype(jnp.float32)
    run = pl.kernel(
        _rpe_body,
        out_type=jax.ShapeDtypeStruct((S, S), jnp.float32),
        mesh=plsc.VectorSubcoreMesh(core_axis_name="c", subcore_axis_name="s"),
        compiler_params=pltpu.CompilerParams(needs_layout_passes=False),
        scratch_types=(
            [pltpu.VMEM((W,), jnp.int32) for _ in range(8)]
            + [pltpu.VMEM((8, W), jnp.float32),
               pltpu.VMEM((NUM_BUCKETS,), jnp.float32),
               pltpu.SemaphoreType.DMA]
        ),
    )
    return run(jnp.asarray(_BT).reshape(-1), w)


# speedup vs baseline: 1.9880x; 1.1654x over previous
import math

import jax
import jax.numpy as jnp
import numpy as np
from jax import lax
from jax.experimental import pallas as pl
from jax.experimental.pallas import tpu as pltpu
from jax.experimental.pallas import tpu_sc as plsc

S = 4096
NUM_BUCKETS = 32
MAX_DISTANCE = 4096
NW = 32
W = 6144
GW = 8320
LANES = 16
NSTRIPE = 16


def _bucket_table() -> np.ndarray:
    g = np.arange(GW, dtype=np.int64)
    rows = []
    for c in range(8):
        d = np.clip(g + c - (S - 1), -(S - 1), S - 1)
        a = np.abs(d)
        safe = np.maximum(a, 1).astype(np.float32)
        log_term = 8.0 + np.ceil(
            np.log(safe / 8.0) / math.log(MAX_DISTANCE / 8.0) * 8.0
        )
        large = np.minimum(np.float32(15.0), log_term).astype(np.int32)
        b = np.where(a < 8, a, large).astype(np.int32)
        rows.append(np.where(d < 0, b + 16, b).astype(np.int32))
    return np.stack(rows)


_BT = _bucket_table()


NCHUNK = 376
QCOL = 1024


def _rpe_body(bt_hbm, w_hbm, out_hbm, *scratch):
    bt_v = scratch[0:8]
    b_v, w_v = scratch[8], scratch[9]
    sem_in, sem_out = scratch[10], scratch[11]
    cid = lax.axis_index("c")
    sid = lax.axis_index("s")
    wid = sid * 2 + cid
    a = wid % 16
    h = wid // 16
    gb = (S - 8) - 8 * a - 128 * (NSTRIPE - 1) - 2048 * h

    stage = [pltpu.async_copy(w_hbm, w_v, sem_in)] + [
        pltpu.async_copy(bt_hbm.at[pl.ds(c * GW + gb, W)], bt_v[c], sem_in)
        for c in range(8)
    ]
    for cp in stage:
        cp.wait()

    def gather_chunk(k, carry):
        for c in range(8):
            idx = bt_v[c][pl.ds(k * LANES, LANES)]
            b_v[7 - c, pl.ds(k * LANES, LANES)] = plsc.load_gather(w_v, [idx])
        return carry

    pieces = sorted(
        (8 * (NSTRIPE - 1 - kk) + (QCOL // LANES) * (q + 1), kk, q)
        for kk in range(NSTRIPE)
        for q in range(S // QCOL)
    )
    handles = []
    cur = 0
    for rc, kk, q in pieces:
        if rc > cur:
            lax.fori_loop(cur, rc, gather_chunk, 0)
            cur = rc
        handles.append(
            pltpu.async_copy(
                b_v.at[:, pl.ds(128 * (NSTRIPE - 1 - kk) + QCOL * q, QCOL)],
                out_hbm.at[
                    pl.ds(8 * a + 2048 * h + 128 * kk, 8),
                    pl.ds(QCOL * q, QCOL),
                ],
                sem_out,
            )
        )
    for hd in handles:
        hd.wait()


def kernel(seq_len, weight):
    del seq_len
    w = weight.reshape(NUM_BUCKETS).astype(jnp.float32)
    run = pl.kernel(
        _rpe_body,
        out_type=jax.ShapeDtypeStruct((S, S), jnp.float32),
        mesh=plsc.VectorSubcoreMesh(core_axis_name="c", subcore_axis_name="s"),
        compiler_params=pltpu.CompilerParams(needs_layout_passes=False),
        scratch_types=(
            [pltpu.VMEM((W,), jnp.int32) for _ in range(8)]
            + [pltpu.VMEM((8, W), jnp.float32),
               pltpu.VMEM((NUM_BUCKETS,), jnp.float32),
               pltpu.SemaphoreType.DMA,
               pltpu.SemaphoreType.DMA]
        ),
    )
    return run(jnp.asarray(_BT).reshape(-1), w)


# speedup vs baseline: 2.1551x; 1.0841x over previous
import math

import jax
import jax.numpy as jnp
import numpy as np
from jax import lax
from jax.experimental import pallas as pl
from jax.experimental.pallas import tpu as pltpu
from jax.experimental.pallas import tpu_sc as plsc

S = 4096
NUM_BUCKETS = 32
MAX_DISTANCE = 4096
NW = 32
W = 6144
GW = 8320
LANES = 16
NSTRIPE = 16


def _bucket_table() -> np.ndarray:
    g = np.arange(GW, dtype=np.int64)
    rows = []
    for c in range(8):
        d = np.clip(g + c - (S - 1), -(S - 1), S - 1)
        a = np.abs(d)
        safe = np.maximum(a, 1).astype(np.float32)
        log_term = 8.0 + np.ceil(
            np.log(safe / 8.0) / math.log(MAX_DISTANCE / 8.0) * 8.0
        )
        large = np.minimum(np.float32(15.0), log_term).astype(np.int32)
        b = np.where(a < 8, a, large).astype(np.int32)
        rows.append(np.where(d < 0, b + 16, b).astype(np.int32))
    return np.stack(rows)


_BT = _bucket_table()


NCHUNK = 376
QCOL = 1024


def _rpe_body(bt_hbm, w_hbm, out_hbm, *scratch):
    bt_v = scratch[0:8]
    b_v, w_v = scratch[8], scratch[9]
    sem_in, sem_out = scratch[10], scratch[11]
    cid = lax.axis_index("c")
    sid = lax.axis_index("s")
    wid = sid * 2 + cid
    a = wid % 16
    h = wid // 16
    gb = (S - 8) - 8 * a - 128 * (NSTRIPE - 1) - 2048 * h

    stage = [pltpu.async_copy(w_hbm, w_v, sem_in)] + [
        pltpu.async_copy(bt_hbm.at[pl.ds(c * GW + gb, W)], bt_v[c], sem_in)
        for c in range(8)
    ]
    for cp in stage:
        cp.wait()

    def gather_chunk(k, carry):
        for c in range(8):
            idx = bt_v[c][pl.ds(k * LANES, LANES)]
            b_v[7 - c, pl.ds(k * LANES, LANES)] = plsc.load_gather(w_v, [idx])
        return carry

    def fire(p):
        j = p % NSTRIPE
        q = p // NSTRIPE
        return pltpu.async_copy(
            b_v.at[:, pl.ds(128 * j + QCOL * q, QCOL)],
            out_hbm.at[
                pl.ds(8 * a + 2048 * h + 128 * (NSTRIPE - 1 - j), 8),
                pl.ds(QCOL * q, QCOL),
            ],
            sem_out,
        )

    nslots = NCHUNK // 8
    npieces = NSTRIPE * (S // QCOL)

    def slot_body(p, carry):
        def slot_chunks(u, c2):
            return gather_chunk(8 * p + u, c2)

        lax.fori_loop(0, 8, slot_chunks, 0)

        @pl.when(p >= 7)
        def _():
            fire(p - 7)

        return carry

    lax.fori_loop(0, nslots, slot_body, 0)

    def late_fire(i, carry):
        fire(nslots - 7 + i)
        return carry

    lax.fori_loop(0, npieces - (nslots - 7), late_fire, 0)

    def drain(i, carry):
        pltpu.make_async_copy(
            b_v.at[:, pl.ds(0, QCOL)],
            out_hbm.at[pl.ds(0, 8), pl.ds(0, QCOL)],
            sem_out,
        ).wait()
        return carry

    lax.fori_loop(0, npieces, drain, 0)


def kernel(seq_len, weight):
    del seq_len
    w = weight.reshape(NUM_BUCKETS).astype(jnp.float32)
    run = pl.kernel(
        _rpe_body,
        out_type=jax.ShapeDtypeStruct((S, S), jnp.float32),
        mesh=plsc.VectorSubcoreMesh(core_axis_name="c", subcore_axis_name="s"),
        compiler_params=pltpu.CompilerParams(needs_layout_passes=False),
        scratch_types=(
            [pltpu.VMEM((W,), jnp.int32) for _ in range(8)]
            + [pltpu.VMEM((8, W), jnp.float32),
               pltpu.VMEM((NUM_BUCKETS,), jnp.float32),
               pltpu.SemaphoreType.DMA,
               pltpu.SemaphoreType.DMA]
        ),
    )
    return run(jnp.asarray(_BT).reshape(-1), w)
